# Initial kernel scaffold; baseline (speedup 1.0000x reference)
import jax
import jax.numpy as jnp
from jax.experimental import pallas as pl


def kernel(X, A, W, b, gamma, beta):
    raise NotImplementedError("write your pallas kernel here")


# speedup vs baseline: 8.0412x; 8.0412x over previous
import functools

import jax
import jax.numpy as jnp
from jax import lax
from jax.experimental import pallas as pl
from jax.experimental.pallas import tpu as pltpu
from jax.experimental.pallas import tpu_sc as plsc

N = 10000
H = 128
CHUNK = 128
NCHUNKS = 1250
NC = 2
NS = 16
ROWS_MAIN = 624
ROWS_TAIL = N - 15 * ROWS_MAIN
CNT_W = 16


def _sc_segment_mean(table, gidx, sidx):
  mesh = plsc.VectorSubcoreMesh(
      core_axis_name="c", subcore_axis_name="s", num_cores=NC,
      num_subcores=NS)

  @functools.partial(
      pl.kernel,
      out_type=jax.ShapeDtypeStruct((NC * N, H), jnp.float32),
      mesh=mesh,
      scratch_types=[
          pltpu.VMEM((CHUNK,), jnp.int32),
          pltpu.VMEM((CHUNK,), jnp.int32),
          pltpu.VMEM((CHUNK, H), jnp.float32),
          pltpu.VMEM((CHUNK,), jnp.float32),
          pltpu.VMEM((ROWS_TAIL,), jnp.float32),
          pltpu.VMEM_SHARED((N, H), jnp.float32),
          pltpu.VMEM_SHARED((N,), jnp.float32),
      ],
  )
  def k(table_hbm, gidx_hbm, sidx_hbm, out_hbm,
        gbuf, sbuf, rows, ones1, cbuf, acc, cnt):
    c = lax.axis_index("c")
    s = lax.axis_index("s")
    base = s * ROWS_MAIN
    coff = c * N
    extra = ROWS_TAIL - ROWS_MAIN

    zv = jnp.zeros((16,), jnp.float32)
    ov = jnp.ones((16,), jnp.float32)
    def zero_row(r, carry):
      for kk in range(H // 16):
        rows[r, pl.ds(kk * 16, 16)] = zv
      return carry
    lax.fori_loop(0, CHUNK, zero_row, 0)
    for kk in range(CHUNK // 16):
      ones1[pl.ds(kk * 16, 16)] = ov
    for kk in range(ROWS_TAIL // 16):
      cbuf[pl.ds(kk * 16, 16)] = zv

    for kb in range(4):
      pltpu.sync_copy(rows, acc.at[pl.ds(base + 128 * kb, 128)])
    pltpu.sync_copy(rows.at[pl.ds(0, ROWS_MAIN - 512)],
                    acc.at[pl.ds(base + 512, ROWS_MAIN - 512)])
    pltpu.sync_copy(cbuf.at[pl.ds(0, ROWS_MAIN)], cnt.at[pl.ds(base, ROWS_MAIN)])

    @pl.when(s == NS - 1)
    def _():
      pltpu.sync_copy(rows.at[pl.ds(0, extra)], acc.at[pl.ds(N - extra, extra)])
      pltpu.sync_copy(cbuf.at[pl.ds(0, extra)], cnt.at[pl.ds(N - extra, extra)])

    plsc.subcore_barrier()

    def chunk_body(t, carry):
      j = s + NS * t

      @pl.when(j < NCHUNKS)
      def _():
        pltpu.sync_copy(gidx_hbm.at[pl.ds(j * CHUNK, CHUNK)], gbuf)
        pltpu.sync_copy(sidx_hbm.at[pl.ds(j * CHUNK, CHUNK)], sbuf)
        for kk in range(CHUNK // 16):
          sl = pl.ds(kk * 16, 16)
          gbuf[sl] = gbuf[sl] + jnp.full((16,), coff, jnp.int32)
        pltpu.sync_copy(table_hbm.at[gbuf], rows)
        pltpu.sync_copy(rows, acc.at[sbuf], add=True)
        pltpu.sync_copy(ones1, cnt.at[sbuf], add=True)

      return carry

    lax.fori_loop(0, (NCHUNKS + NS - 1) // NS, chunk_body, 0)
    plsc.subcore_barrier()

    pltpu.sync_copy(cnt.at[pl.ds(base, ROWS_MAIN)], cbuf.at[pl.ds(0, ROWS_MAIN)])

    @pl.when(s == NS - 1)
    def _():
      pltpu.sync_copy(cnt.at[pl.ds(N - extra, extra)],
                      cbuf.at[pl.ds(ROWS_MAIN, extra)])

    def scale_block(rb, lb, sk):
      pltpu.sync_copy(acc.at[pl.ds(rb, sk)], rows.at[pl.ds(0, sk)])

      def scale_group(g, carry):
        cg = cbuf[pl.ds(lb + g * 16, 16)]
        ig = jnp.where(cg > 0, 1.0 / cg, 0.0)
        for r16 in range(16):
          iv = jnp.take(ig, jnp.full((16,), r16, jnp.int32))
          r = g * 16 + r16
          for kk in range(H // 16):
            sl = pl.ds(kk * 16, 16)
            rows[r, sl] = rows[r, sl] * iv
        return carry

      lax.fori_loop(0, sk // 16, scale_group, 0)
      pltpu.sync_copy(rows.at[pl.ds(0, sk)], out_hbm.at[pl.ds(coff + rb, sk)])

    for kb in range(4):
      scale_block(base + 128 * kb, 128 * kb, 128)
    scale_block(base + 512, 512, ROWS_MAIN - 512)

    @pl.when(s == NS - 1)
    def _():
      scale_block(N - extra, ROWS_MAIN, extra)

  return k(table, gidx, sidx)


_BM = 400


def _tc_epilogue_body(u0_ref, u1_ref, w_ref, b_ref, g_ref, be_ref, o_ref):
  y = jnp.concatenate([u0_ref[...], u1_ref[...]], axis=1)
  z = lax.dot_general(y, w_ref[...], (((1,), (1,)), ((), ())),
                      preferred_element_type=jnp.float32)
  z = z + b_ref[...][None, :]
  z = jnp.where(z >= 0, z, 0.01 * z)
  mu = jnp.mean(z, axis=1, keepdims=True)
  zc = z - mu
  var = jnp.mean(zc * zc, axis=1, keepdims=True)
  o_ref[...] = zc * lax.rsqrt(var + 1e-5) * g_ref[...][None, :] \
      + be_ref[...][None, :]


def _tc_epilogue(u, W, b, gamma, beta):
  nblk = N // _BM
  return pl.pallas_call(
      _tc_epilogue_body,
      out_shape=jax.ShapeDtypeStruct((N, 2 * H), jnp.float32),
      grid=(nblk,),
      in_specs=[
          pl.BlockSpec((_BM, H), lambda i: (i, 0)),
          pl.BlockSpec((_BM, H), lambda i: (i + nblk, 0)),
          pl.BlockSpec((2 * H, 2 * H), lambda i: (0, 0)),
          pl.BlockSpec((2 * H,), lambda i: (0,)),
          pl.BlockSpec((2 * H,), lambda i: (0,)),
          pl.BlockSpec((2 * H,), lambda i: (0,)),
      ],
      out_specs=pl.BlockSpec((_BM, 2 * H), lambda i: (i, 0)),
  )(u, u, W, b, gamma, beta)


def kernel(X, A, W, b, gamma, beta):
  x2 = jnp.concatenate([X[:, :H], X[:, H:]], axis=0)
  ef = _sc_segment_mean(x2, A[0], A[1])
  ou = _sc_segment_mean(ef, A[1], A[0])
  return _tc_epilogue(ou, W, b, gamma, beta)


# speedup vs baseline: 10.3137x; 1.2826x over previous
import functools

import jax
import jax.numpy as jnp
from jax import lax
from jax.experimental import pallas as pl
from jax.experimental.pallas import tpu as pltpu
from jax.experimental.pallas import tpu_sc as plsc

N = 10000
NA = 10240
H = 128
CHUNK = 128
NCHUNKS = 1280
NC = 2
NS = 16
RPT = NA // NS
CPT = NCHUNKS // NS
STAGE = 40


def _sc_segment_mean(table, gidx, sidx):
  mesh = plsc.VectorSubcoreMesh(
      core_axis_name="c", subcore_axis_name="s", num_cores=NC,
      num_subcores=NS)

  @functools.partial(
      pl.kernel,
      out_type=jax.ShapeDtypeStruct((NC * NA, H), jnp.float32),
      mesh=mesh,
      scratch_types=[
          pltpu.VMEM((STAGE, CHUNK), jnp.int32),
          pltpu.VMEM((STAGE, CHUNK), jnp.int32),
          pltpu.VMEM((CHUNK, H), jnp.float32),
          pltpu.VMEM((CHUNK, H), jnp.float32),
          pltpu.VMEM((CHUNK,), jnp.float32),
          pltpu.VMEM((RPT,), jnp.float32),
          pltpu.VMEM_SHARED((NA, H), jnp.float32),
          pltpu.VMEM_SHARED((NA,), jnp.float32),
          pltpu.SemaphoreType.DMA,
          pltpu.SemaphoreType.DMA,
          pltpu.SemaphoreType.DMA,
          pltpu.SemaphoreType.DMA,
          pltpu.SemaphoreType.DMA,
          pltpu.SemaphoreType.DMA,
      ],
  )
  def k(table_hbm, gidx_hbm, sidx_hbm, out_hbm,
        gstage, sstage, rows0, rows1, ones1, cbuf, acc, cnt,
        sg0, sg1, ss0, ss1, sc0, sc1):
    c = lax.axis_index("c")
    s = lax.axis_index("s")
    base = s * RPT
    cbase = s * CPT
    coff = c * NA

    zv = jnp.zeros((16,), jnp.float32)
    ov = jnp.ones((16,), jnp.float32)

    def zero_row(r, carry):
      for kk in range(H // 16):
        rows0[r, pl.ds(kk * 16, 16)] = zv
      return carry

    lax.fori_loop(0, CHUNK, zero_row, 0)
    for kk in range(CHUNK // 16):
      ones1[pl.ds(kk * 16, 16)] = ov
    for kk in range(RPT // 16):
      cbuf[pl.ds(kk * 16, 16)] = zv

    for kb in range(RPT // CHUNK):
      pltpu.sync_copy(rows0, acc.at[pl.ds(base + CHUNK * kb, CHUNK)])
    pltpu.sync_copy(cbuf, cnt.at[pl.ds(base, RPT)])
    plsc.subcore_barrier()

    coffv = jnp.full((16,), coff, jnp.int32)

    for stage in range(CPT // STAGE):
      pltpu.sync_copy(gidx_hbm.at[pl.ds(cbase + STAGE * stage, STAGE)], gstage)
      pltpu.sync_copy(sidx_hbm.at[pl.ds(cbase + STAGE * stage, STAGE)], sstage)

      def adjust_row(q, carry):
        for kk in range(CHUNK // 16):
          sl = pl.ds(kk * 16, 16)
          gstage[q, sl] = gstage[q, sl] + coffv
        return carry

      lax.fori_loop(0, STAGE, adjust_row, 0)

      def pair_body(t2, carry):
        b0 = 2 * t2
        b1 = b0 + 1
        d0 = pltpu.async_copy(table_hbm.at[gstage.at[b0]], rows0, sg0)
        d1 = pltpu.async_copy(table_hbm.at[gstage.at[b1]], rows1, sg1)
        d0.wait()
        e0 = pltpu.async_copy(rows0, acc.at[sstage.at[b0]], ss0, add=True)
        f0 = pltpu.async_copy(ones1, cnt.at[sstage.at[b0]], sc0, add=True)
        d1.wait()
        e1 = pltpu.async_copy(rows1, acc.at[sstage.at[b1]], ss1, add=True)
        f1 = pltpu.async_copy(ones1, cnt.at[sstage.at[b1]], sc1, add=True)
        e0.wait()
        f0.wait()
        e1.wait()
        f1.wait()
        return carry

      lax.fori_loop(0, STAGE // 2, pair_body, 0)

    plsc.subcore_barrier()

    pltpu.sync_copy(cnt.at[pl.ds(base, RPT)], cbuf)

    def scale_block(rb, lb):
      pltpu.sync_copy(acc.at[pl.ds(rb, CHUNK)], rows0)

      def scale_group(g, carry):
        cg = cbuf[pl.ds(lb + g * 16, 16)]
        ig = jnp.where(cg > 0, 1.0 / cg, 0.0)
        for r16 in range(16):
          iv = jnp.take(ig, jnp.full((16,), r16, jnp.int32))
          r = g * 16 + r16
          for kk in range(H // 16):
            sl = pl.ds(kk * 16, 16)
            rows0[r, sl] = rows0[r, sl] * iv
        return carry

      lax.fori_loop(0, CHUNK // 16, scale_group, 0)
      pltpu.sync_copy(rows0, out_hbm.at[pl.ds(coff + rb, CHUNK)])

    for kb in range(RPT // CHUNK):
      scale_block(base + CHUNK * kb, CHUNK * kb)

  return k(table, gidx, sidx)


_BM = 80


def _tc_epilogue_body(u0_ref, u1_ref, w_ref, b_ref, g_ref, be_ref, o_ref):
  y = jnp.concatenate([u0_ref[...], u1_ref[...]], axis=1)
  z = lax.dot_general(y, w_ref[...], (((1,), (1,)), ((), ())),
                      preferred_element_type=jnp.float32)
  z = z + b_ref[...][None, :]
  z = jnp.where(z >= 0, z, 0.01 * z)
  mu = jnp.mean(z, axis=1, keepdims=True)
  zc = z - mu
  var = jnp.mean(zc * zc, axis=1, keepdims=True)
  o_ref[...] = zc * lax.rsqrt(var + 1e-5) * g_ref[...][None, :] \
      + be_ref[...][None, :]


def _tc_epilogue(u, W, b, gamma, beta):
  nblk1 = NA // _BM
  return pl.pallas_call(
      _tc_epilogue_body,
      out_shape=jax.ShapeDtypeStruct((N, 2 * H), jnp.float32),
      grid=(N // _BM,),
      in_specs=[
          pl.BlockSpec((_BM, H), lambda i: (i, 0)),
          pl.BlockSpec((_BM, H), lambda i: (i + nblk1, 0)),
          pl.BlockSpec((2 * H, 2 * H), lambda i: (0, 0)),
          pl.BlockSpec((2 * H,), lambda i: (0,)),
          pl.BlockSpec((2 * H,), lambda i: (0,)),
          pl.BlockSpec((2 * H,), lambda i: (0,)),
      ],
      out_specs=pl.BlockSpec((_BM, 2 * H), lambda i: (i, 0)),
  )(u, u, W, b, gamma, beta)


def kernel(X, A, W, b, gamma, beta):
  zpad = jnp.zeros((NA - N, H), jnp.float32)
  x2 = jnp.concatenate([X[:, :H], zpad, X[:, H:], zpad], axis=0)
  npad = NCHUNKS * CHUNK - A.shape[1]
  trash = N + (jnp.arange(npad, dtype=jnp.int32) % 16)
  A_pad = jnp.concatenate([A, jnp.stack([trash, trash])], axis=1)
  gidx1 = A_pad[0].reshape(NCHUNKS, CHUNK)
  gidx2 = A_pad[1].reshape(NCHUNKS, CHUNK)
  ef = _sc_segment_mean(x2, gidx1, gidx2)
  ou = _sc_segment_mean(ef, gidx2, gidx1)
  return _tc_epilogue(ou, W, b, gamma, beta)


# speedup vs baseline: 11.9572x; 1.1593x over previous
import functools

import jax
import jax.numpy as jnp
from jax import lax
from jax.experimental import pallas as pl
from jax.experimental.pallas import tpu as pltpu
from jax.experimental.pallas import tpu_sc as plsc

N = 10000
NA = 10240
H = 128
CHUNK = 128
NCHUNKS = 1280
NC = 2
NS = 16
RPT = NA // NS
CPT = NCHUNKS // NS
STAGE = 40


def _sc_segment_mean(table, gidx, sidx, keep_trash):
  n_out = NA if keep_trash else N
  mesh = plsc.VectorSubcoreMesh(
      core_axis_name="c", subcore_axis_name="s", num_cores=NC,
      num_subcores=NS)

  @functools.partial(
      pl.kernel,
      out_type=jax.ShapeDtypeStruct((NC * n_out, H), jnp.float32),
      mesh=mesh,
      scratch_types=[
          pltpu.VMEM((STAGE, CHUNK), jnp.int32),
          pltpu.VMEM((STAGE, CHUNK), jnp.int32),
          pltpu.VMEM((CHUNK, H), jnp.float32),
          pltpu.VMEM((CHUNK, H), jnp.float32),
          pltpu.VMEM((CHUNK,), jnp.float32),
          pltpu.VMEM((RPT,), jnp.float32),
          pltpu.VMEM_SHARED((NA, H), jnp.float32),
          pltpu.VMEM_SHARED((NA,), jnp.float32),
          pltpu.SemaphoreType.DMA,
          pltpu.SemaphoreType.DMA,
          pltpu.SemaphoreType.DMA,
          pltpu.SemaphoreType.DMA,
          pltpu.SemaphoreType.DMA,
          pltpu.SemaphoreType.DMA,
      ],
  )
  def k(table_hbm, gidx_hbm, sidx_hbm, out_hbm,
        gstage, sstage, rows0, rows1, ones1, cbuf, acc, cnt,
        sg0, sg1, ss0, ss1, sc0, sc1):
    c = lax.axis_index("c")
    s = lax.axis_index("s")
    base = s * RPT
    cbase = s * CPT
    coff = c * NA

    zv = jnp.zeros((16,), jnp.float32)
    ov = jnp.ones((16,), jnp.float32)

    def zero_row(r, carry):
      for kk in range(H // 16):
        rows0[r, pl.ds(kk * 16, 16)] = zv
      return carry

    lax.fori_loop(0, CHUNK, zero_row, 0)
    for kk in range(CHUNK // 16):
      ones1[pl.ds(kk * 16, 16)] = ov
    for kk in range(RPT // 16):
      cbuf[pl.ds(kk * 16, 16)] = zv

    for kb in range(RPT // CHUNK):
      pltpu.sync_copy(rows0, acc.at[pl.ds(base + CHUNK * kb, CHUNK)])
    pltpu.sync_copy(cbuf, cnt.at[pl.ds(base, RPT)])
    plsc.subcore_barrier()

    coffv = jnp.full((16,), coff, jnp.int32)

    for stage in range(CPT // STAGE):
      pltpu.sync_copy(gidx_hbm.at[pl.ds(cbase + STAGE * stage, STAGE)], gstage)
      pltpu.sync_copy(sidx_hbm.at[pl.ds(cbase + STAGE * stage, STAGE)], sstage)

      def adjust_row(q, carry):
        for kk in range(CHUNK // 16):
          sl = pl.ds(kk * 16, 16)
          gstage[q, sl] = gstage[q, sl] + coffv
        return carry

      lax.fori_loop(0, STAGE, adjust_row, 0)

      def pair_body(t2, carry):
        b0 = 2 * t2
        b1 = b0 + 1

        @pl.when(t2 > 0)
        def _():
          pltpu.make_async_copy(rows0, acc.at[sstage.at[b0]], ss0).wait()
          pltpu.make_async_copy(ones1, cnt.at[sstage.at[b0]], sc0).wait()
          pltpu.make_async_copy(rows1, acc.at[sstage.at[b1]], ss1).wait()
          pltpu.make_async_copy(ones1, cnt.at[sstage.at[b1]], sc1).wait()

        d0 = pltpu.async_copy(table_hbm.at[gstage.at[b0]], rows0, sg0)
        d1 = pltpu.async_copy(table_hbm.at[gstage.at[b1]], rows1, sg1)
        d0.wait()
        pltpu.async_copy(rows0, acc.at[sstage.at[b0]], ss0, add=True)
        pltpu.async_copy(ones1, cnt.at[sstage.at[b0]], sc0, add=True)
        d1.wait()
        pltpu.async_copy(rows1, acc.at[sstage.at[b1]], ss1, add=True)
        pltpu.async_copy(ones1, cnt.at[sstage.at[b1]], sc1, add=True)
        return carry

      lax.fori_loop(0, STAGE // 2, pair_body, 0)
      pltpu.make_async_copy(rows0, acc.at[sstage.at[STAGE - 2]], ss0).wait()
      pltpu.make_async_copy(ones1, cnt.at[sstage.at[STAGE - 2]], sc0).wait()
      pltpu.make_async_copy(rows1, acc.at[sstage.at[STAGE - 1]], ss1).wait()
      pltpu.make_async_copy(ones1, cnt.at[sstage.at[STAGE - 1]], sc1).wait()

    plsc.subcore_barrier()

    pltpu.sync_copy(cnt.at[pl.ds(base, RPT)], cbuf)

    coff_o = c * n_out

    def scale_block(rb, lb, kb):
      pltpu.sync_copy(acc.at[pl.ds(rb, CHUNK)], rows0)

      def scale_group(g, carry):
        cg = cbuf[pl.ds(lb + g * 16, 16)]
        ig = jnp.where(cg > 0, 1.0 / cg, 0.0)
        for r16 in range(16):
          iv = jnp.take(ig, jnp.full((16,), r16, jnp.int32))
          r = g * 16 + r16
          for kk in range(H // 16):
            sl = pl.ds(kk * 16, 16)
            rows0[r, sl] = rows0[r, sl] * iv
        return carry

      lax.fori_loop(0, CHUNK // 16, scale_group, 0)
      if keep_trash:
        pltpu.sync_copy(rows0, out_hbm.at[pl.ds(coff_o + rb, CHUNK)])
      else:
        @pl.when(s < NS - 1)
        def _():
          pltpu.sync_copy(rows0, out_hbm.at[pl.ds(coff_o + rb, CHUNK)])

        @pl.when(s == NS - 1)
        def _():
          if kb < 3:
            pltpu.sync_copy(rows0, out_hbm.at[pl.ds(coff_o + rb, CHUNK)])
          elif kb == 3:
            last = N - (NS - 1) * RPT - 3 * CHUNK
            pltpu.sync_copy(rows0.at[pl.ds(0, last)],
                            out_hbm.at[pl.ds(coff_o + rb, last)])

    for kb in range(RPT // CHUNK):
      scale_block(base + CHUNK * kb, CHUNK * kb, kb)

  return k(table, gidx, sidx)


_BM = 400


def _tc_epilogue_body(u0_ref, u1_ref, w_ref, b_ref, g_ref, be_ref, o_ref):
  y = jnp.concatenate([u0_ref[...], u1_ref[...]], axis=1)
  z = lax.dot_general(y, w_ref[...], (((1,), (1,)), ((), ())),
                      preferred_element_type=jnp.float32)
  z = z + b_ref[...][None, :]
  z = jnp.where(z >= 0, z, 0.01 * z)
  mu = jnp.mean(z, axis=1, keepdims=True)
  zc = z - mu
  var = jnp.mean(zc * zc, axis=1, keepdims=True)
  o_ref[...] = zc * lax.rsqrt(var + 1e-5) * g_ref[...][None, :] \
      + be_ref[...][None, :]


def _tc_epilogue(u, W, b, gamma, beta):
  nblk1 = N // _BM
  return pl.pallas_call(
      _tc_epilogue_body,
      out_shape=jax.ShapeDtypeStruct((N, 2 * H), jnp.float32),
      grid=(N // _BM,),
      in_specs=[
          pl.BlockSpec((_BM, H), lambda i: (i, 0)),
          pl.BlockSpec((_BM, H), lambda i: (i + nblk1, 0)),
          pl.BlockSpec((2 * H, 2 * H), lambda i: (0, 0)),
          pl.BlockSpec((2 * H,), lambda i: (0,)),
          pl.BlockSpec((2 * H,), lambda i: (0,)),
          pl.BlockSpec((2 * H,), lambda i: (0,)),
      ],
      out_specs=pl.BlockSpec((_BM, 2 * H), lambda i: (i, 0)),
  )(u, u, W, b, gamma, beta)


def kernel(X, A, W, b, gamma, beta):
  zpad = jnp.zeros((NA - N, H), jnp.float32)
  x2 = jnp.concatenate([X[:, :H], zpad, X[:, H:], zpad], axis=0)
  npad = NCHUNKS * CHUNK - A.shape[1]
  trash = N + (jnp.arange(npad, dtype=jnp.int32) % 16)
  A_pad = jnp.concatenate([A, jnp.stack([trash, trash])], axis=1)
  gidx1 = A_pad[0].reshape(NCHUNKS, CHUNK)
  gidx2 = A_pad[1].reshape(NCHUNKS, CHUNK)
  ef = _sc_segment_mean(x2, gidx1, gidx2, keep_trash=True)
  ou = _sc_segment_mean(ef, gidx2, gidx1, keep_trash=False)
  return _tc_epilogue(ou, W, b, gamma, beta)


# speedup vs baseline: 12.1185x; 1.0135x over previous
import functools

import jax
import jax.numpy as jnp
from jax import lax
from jax.experimental import pallas as pl
from jax.experimental.pallas import tpu as pltpu
from jax.experimental.pallas import tpu_sc as plsc

N = 10000
NA = 10240
H = 128
CHUNK = 128
NCHUNKS = 1280
NC = 2
NS = 16
RPT = NA // NS
CPT = NCHUNKS // NS
STAGE = 40


def _sc_propagate(table, gidx1, gidx2):
  mesh = plsc.VectorSubcoreMesh(
      core_axis_name="c", subcore_axis_name="s", num_cores=NC,
      num_subcores=NS)

  @functools.partial(
      pl.kernel,
      out_type=(jax.ShapeDtypeStruct((NC * NA, H), jnp.float32),
                jax.ShapeDtypeStruct((NC * N, H), jnp.float32)),
      mesh=mesh,
      scratch_types=[
          pltpu.VMEM((STAGE, CHUNK), jnp.int32),
          pltpu.VMEM((STAGE, CHUNK), jnp.int32),
          pltpu.VMEM((CHUNK, H), jnp.float32),
          pltpu.VMEM((CHUNK, H), jnp.float32),
          pltpu.VMEM((CHUNK,), jnp.float32),
          pltpu.VMEM((RPT,), jnp.float32),
          pltpu.VMEM_SHARED((NA, H), jnp.float32),
          pltpu.VMEM_SHARED((NA,), jnp.float32),
          pltpu.SemaphoreType.DMA,
          pltpu.SemaphoreType.DMA,
          pltpu.SemaphoreType.DMA,
          pltpu.SemaphoreType.DMA,
          pltpu.SemaphoreType.DMA,
          pltpu.SemaphoreType.DMA,
      ],
  )
  def k(table_hbm, g1_hbm, g2_hbm, ef_hbm, out_hbm,
        gstage, sstage, rows0, rows1, ones1, cbuf, acc, cnt,
        sg0, sg1, ss0, ss1, sc0, sc1):
    c = lax.axis_index("c")
    s = lax.axis_index("s")
    base = s * RPT
    cbase = s * CPT
    coff = c * NA
    coffv = jnp.full((16,), coff, jnp.int32)
    zv = jnp.zeros((16,), jnp.float32)
    ov = jnp.ones((16,), jnp.float32)

    def zero_acc_slices():
      def zero_row(r, carry):
        for kk in range(H // 16):
          rows0[r, pl.ds(kk * 16, 16)] = zv
        return carry

      lax.fori_loop(0, CHUNK, zero_row, 0)
      for kk in range(RPT // 16):
        cbuf[pl.ds(kk * 16, 16)] = zv
      for kb in range(RPT // CHUNK):
        pltpu.sync_copy(rows0, acc.at[pl.ds(base + CHUNK * kb, CHUNK)])
      pltpu.sync_copy(cbuf, cnt.at[pl.ds(base, RPT)])

    def chunk_loop(tbl, gidx_hbm, sidx_hbm):
      for stage in range(CPT // STAGE):
        pltpu.sync_copy(gidx_hbm.at[pl.ds(cbase + STAGE * stage, STAGE)],
                        gstage)
        pltpu.sync_copy(sidx_hbm.at[pl.ds(cbase + STAGE * stage, STAGE)],
                        sstage)

        def adjust_row(q, carry):
          for kk in range(CHUNK // 16):
            sl = pl.ds(kk * 16, 16)
            gstage[q, sl] = gstage[q, sl] + coffv
          return carry

        lax.fori_loop(0, STAGE, adjust_row, 0)

        def pair_body(t2, carry):
          b0 = 2 * t2
          b1 = b0 + 1

          @pl.when(t2 > 0)
          def _():
            pltpu.make_async_copy(rows0, acc.at[sstage.at[b0]], ss0).wait()
            pltpu.make_async_copy(ones1, cnt.at[sstage.at[b0]], sc0).wait()
            pltpu.make_async_copy(rows1, acc.at[sstage.at[b1]], ss1).wait()
            pltpu.make_async_copy(ones1, cnt.at[sstage.at[b1]], sc1).wait()

          d0 = pltpu.async_copy(tbl.at[gstage.at[b0]], rows0, sg0)
          d1 = pltpu.async_copy(tbl.at[gstage.at[b1]], rows1, sg1)
          d0.wait()
          pltpu.async_copy(rows0, acc.at[sstage.at[b0]], ss0, add=True)
          pltpu.async_copy(ones1, cnt.at[sstage.at[b0]], sc0, add=True)
          d1.wait()
          pltpu.async_copy(rows1, acc.at[sstage.at[b1]], ss1, add=True)
          pltpu.async_copy(ones1, cnt.at[sstage.at[b1]], sc1, add=True)
          return carry

        lax.fori_loop(0, STAGE // 2, pair_body, 0)
        pltpu.make_async_copy(rows0, acc.at[sstage.at[STAGE - 2]], ss0).wait()
        pltpu.make_async_copy(ones1, cnt.at[sstage.at[STAGE - 2]], sc0).wait()
        pltpu.make_async_copy(rows1, acc.at[sstage.at[STAGE - 1]], ss1).wait()
        pltpu.make_async_copy(ones1, cnt.at[sstage.at[STAGE - 1]], sc1).wait()

    def epilogue(out_ref, keep_trash):
      pltpu.sync_copy(cnt.at[pl.ds(base, RPT)], cbuf)
      coff_o = c * (NA if keep_trash else N)

      def scale_block(rb, lb, kb):
        pltpu.sync_copy(acc.at[pl.ds(rb, CHUNK)], rows0)

        def scale_group(g, carry):
          cg = cbuf[pl.ds(lb + g * 16, 16)]
          ig = jnp.where(cg > 0, 1.0 / cg, 0.0)
          for r16 in range(16):
            iv = jnp.take(ig, jnp.full((16,), r16, jnp.int32))
            r = g * 16 + r16
            for kk in range(H // 16):
              sl = pl.ds(kk * 16, 16)
              rows0[r, sl] = rows0[r, sl] * iv
          return carry

        lax.fori_loop(0, CHUNK // 16, scale_group, 0)
        if keep_trash:
          pltpu.sync_copy(rows0, out_ref.at[pl.ds(coff_o + rb, CHUNK)])
        else:
          @pl.when(s < NS - 1)
          def _():
            pltpu.sync_copy(rows0, out_ref.at[pl.ds(coff_o + rb, CHUNK)])

          @pl.when(s == NS - 1)
          def _():
            if kb < 3:
              pltpu.sync_copy(rows0, out_ref.at[pl.ds(coff_o + rb, CHUNK)])
            elif kb == 3:
              last = N - (NS - 1) * RPT - 3 * CHUNK
              pltpu.sync_copy(rows0.at[pl.ds(0, last)],
                              out_ref.at[pl.ds(coff_o + rb, last)])

      for kb in range(RPT // CHUNK):
        scale_block(base + CHUNK * kb, CHUNK * kb, kb)

    zero_acc_slices()
    for kk in range(CHUNK // 16):
      ones1[pl.ds(kk * 16, 16)] = ov
    plsc.subcore_barrier()
    chunk_loop(table_hbm, g1_hbm, g2_hbm)
    plsc.subcore_barrier()
    epilogue(ef_hbm, keep_trash=True)

    zero_acc_slices()
    plsc.subcore_barrier()
    chunk_loop(ef_hbm, g2_hbm, g1_hbm)
    plsc.subcore_barrier()
    epilogue(out_hbm, keep_trash=False)

  return k(table, gidx1, gidx2)[1]


_BM = 400


def _tc_epilogue_body(u0_ref, u1_ref, w_ref, b_ref, g_ref, be_ref, o_ref):
  y = jnp.concatenate([u0_ref[...], u1_ref[...]], axis=1)
  z = lax.dot_general(y, w_ref[...], (((1,), (1,)), ((), ())),
                      preferred_element_type=jnp.float32)
  z = z + b_ref[...][None, :]
  z = jnp.where(z >= 0, z, 0.01 * z)
  mu = jnp.mean(z, axis=1, keepdims=True)
  zc = z - mu
  var = jnp.mean(zc * zc, axis=1, keepdims=True)
  o_ref[...] = zc * lax.rsqrt(var + 1e-5) * g_ref[...][None, :] \
      + be_ref[...][None, :]


def _tc_epilogue(u, W, b, gamma, beta):
  nblk1 = N // _BM
  return pl.pallas_call(
      _tc_epilogue_body,
      out_shape=jax.ShapeDtypeStruct((N, 2 * H), jnp.float32),
      grid=(N // _BM,),
      in_specs=[
          pl.BlockSpec((_BM, H), lambda i: (i, 0)),
          pl.BlockSpec((_BM, H), lambda i: (i + nblk1, 0)),
          pl.BlockSpec((2 * H, 2 * H), lambda i: (0, 0)),
          pl.BlockSpec((2 * H,), lambda i: (0,)),
          pl.BlockSpec((2 * H,), lambda i: (0,)),
          pl.BlockSpec((2 * H,), lambda i: (0,)),
      ],
      out_specs=pl.BlockSpec((_BM, 2 * H), lambda i: (i, 0)),
  )(u, u, W, b, gamma, beta)


def kernel(X, A, W, b, gamma, beta):
  zpad = jnp.zeros((NA - N, H), jnp.float32)
  x2 = jnp.concatenate([X[:, :H], zpad, X[:, H:], zpad], axis=0)
  npad = NCHUNKS * CHUNK - A.shape[1]
  trash = N + (jnp.arange(npad, dtype=jnp.int32) % 16)
  A_pad = jnp.concatenate([A, jnp.stack([trash, trash])], axis=1)
  gidx1 = A_pad[0].reshape(NCHUNKS, CHUNK)
  gidx2 = A_pad[1].reshape(NCHUNKS, CHUNK)
  ou = _sc_propagate(x2, gidx1, gidx2)
  return _tc_epilogue(ou, W, b, gamma, beta)
